# baseline (speedup 1.0000x reference)
import functools

import jax
import jax.numpy as jnp
from jax import lax
from jax.experimental import pallas as pl
from jax.experimental.pallas import tpu as pltpu
from jax.experimental.pallas import tpu_sc as plsc

_D = 64
_B = 4096
_L = 200
_LP = 208
_REGS = 1e-05

_ROWS_PER_STEP = 2000


def _transform_body(x_ref, w_ref, b_ref, o_ref):
    o_ref[...] = lax.dot_general(
        x_ref[...], w_ref[...], (((1,), (1,)), ((), ())),
        preferred_element_type=jnp.float32) + b_ref[...]


def _transform_table(table, w, b2d):
    n = table.shape[0]
    return pl.pallas_call(
        _transform_body,
        grid=(n // _ROWS_PER_STEP,),
        in_specs=[
            pl.BlockSpec((_ROWS_PER_STEP, _D), lambda i: (i, 0)),
            pl.BlockSpec((_D, _D), lambda i: (0, 0)),
            pl.BlockSpec((1, _D), lambda i: (0, 0)),
        ],
        out_specs=pl.BlockSpec((_ROWS_PER_STEP, _D), lambda i: (i, 0)),
        out_shape=jax.ShapeDtypeStruct((n, _D), jnp.float32),
    )(table, w, b2d)


_NC = 2
_NS = 16
_NW = _NC * _NS
_TPW = _B // _NW
_IDXW = _TPW * _L
_NCH = _LP // 16


def _sc_dist(t_item, t_user, items_flat, user):
    mesh = plsc.VectorSubcoreMesh(core_axis_name="c", subcore_axis_name="s")

    @functools.partial(
        pl.kernel,
        out_type=(jax.ShapeDtypeStruct((_B * _LP,), jnp.float32),
                  jax.ShapeDtypeStruct((_B * _LP,), jnp.float32),
                  jax.ShapeDtypeStruct((_B, _D), jnp.float32)),
        mesh=mesh,
        scratch_types=[
            pltpu.VMEM((_IDXW,), jnp.int32),
            pltpu.VMEM((_TPW,), jnp.int32),
            pltpu.VMEM((_TPW, _D), jnp.float32),
            pltpu.VMEM((_LP, _D), jnp.float32),
            pltpu.VMEM((_LP, _D), jnp.float32),
            pltpu.VMEM((_TPW * _LP,), jnp.float32),
            pltpu.VMEM((_TPW * _LP,), jnp.float32),
            pltpu.SemaphoreType.DMA,
            pltpu.SemaphoreType.DMA,
            pltpu.SemaphoreType.DMA,
        ],
        compiler_params=pltpu.CompilerParams(
            use_tc_tiling_on_sc=False, needs_layout_passes=False),
    )
    def k(t_item_hbm, t_user_hbm, items_hbm, user_hbm,
          d2_hbm, n_hbm, ue_hbm,
          idx_v, uidx_v, ue_v, rows0, rows1, d2_v, n_v, sem0, sem1, semu):
        wid = lax.axis_index("s") * _NC + lax.axis_index("c")
        tb = pl.multiple_of(wid * _TPW, 8)
        ib = pl.multiple_of(wid * _IDXW, 8)
        ob = pl.multiple_of(wid * (_TPW * _LP), 8)
        iota = lax.iota(jnp.int32, 16)

        pltpu.sync_copy(items_hbm.at[pl.ds(ib, _IDXW)], idx_v)
        pltpu.sync_copy(user_hbm.at[pl.ds(tb, _TPW)], uidx_v)
        pltpu.async_copy(t_user_hbm.at[uidx_v], ue_v, semu).wait()

        zz = jnp.zeros((16,), jnp.float32)
        for buf in (rows0, rows1):
            for r in range(_L, _LP):
                for c in range(_D // 16):
                    plsc.store_scatter(
                        buf, [jnp.full((16,), r, jnp.int32), c * 16 + iota], zz)

        def copies(u, buf, sem):
            base = pl.multiple_of(u * _L, 8)
            return (
                pltpu.make_async_copy(
                    t_item_hbm.at[idx_v.at[pl.ds(base, 128)]],
                    buf.at[pl.ds(0, 128)], sem),
                pltpu.make_async_copy(
                    t_item_hbm.at[idx_v.at[pl.ds(base + 128, _L - 128)]],
                    buf.at[pl.ds(128, _L - 128)], sem),
            )

        def issue(u, buf, sem):
            for c in copies(u, buf, sem):
                c.start()

        def wait(u, buf, sem):
            for c in copies(u, buf, sem):
                c.wait()

        def compute(u, buf):
            ufull = jnp.full((16,), u, jnp.int32)

            def dbody(d, accs):
                dfull = jnp.full((16,), d, jnp.int32)
                ub = plsc.load_gather(ue_v, [ufull, dfull])
                out = list(accs)
                for lc in range(_NCH):
                    gv = plsc.load_gather(buf, [lc * 16 + iota, dfull])
                    df = gv - ub
                    out[lc] = accs[lc] + df * df
                    out[_NCH + lc] = accs[_NCH + lc] + gv * gv
                return tuple(out)

            accs = lax.fori_loop(
                0, _D, dbody,
                tuple(jnp.zeros((16,), jnp.float32) for _ in range(2 * _NCH)))
            for lc in range(_NCH):
                off = pl.multiple_of(u * _LP + lc * 16, 8)
                d2_v[pl.ds(off, 16)] = accs[lc]
                n_v[pl.ds(off, 16)] = accs[_NCH + lc]

        issue(0, rows0, sem0)

        def pair(p, carry):
            for b in (0, 1):
                buf, sem = (rows0, sem0) if b == 0 else (rows1, sem1)
                nbuf, nsem = (rows1, sem1) if b == 0 else (rows0, sem0)
                u = p * 2 + b

                @pl.when(u + 1 < _TPW)
                def _():
                    issue(u + 1, nbuf, nsem)

                wait(u, buf, sem)
                compute(u, buf)
            return carry

        lax.fori_loop(0, _TPW // 2, pair, 0)

        pltpu.sync_copy(d2_v, d2_hbm.at[pl.ds(ob, _TPW * _LP)])
        pltpu.sync_copy(n_v, n_hbm.at[pl.ds(ob, _TPW * _LP)])
        pltpu.sync_copy(ue_v, ue_hbm.at[pl.ds(tb, _TPW)])

    return k(t_item, t_user, items_flat, user)


_UB = 512
_GSTEPS = _B // _UB


def _loss_body(d2_ref, n_ref, rew_ref, gum_ref, ue_ref, gan_ref, reg_ref):
    step = pl.program_id(0)
    iota_p = lax.broadcasted_iota(jnp.int32, (_UB, _LP), 1)
    mask = iota_p < _L
    dist = jnp.sqrt(d2_ref[...] + 1e-12)
    m = jnp.max(jnp.where(mask, dist, -jnp.inf), axis=-1, keepdims=True)
    sh = dist - m
    ex = jnp.where(mask, jnp.exp(sh), 0.0)
    se = jnp.sum(ex, axis=-1, keepdims=True)
    probs = ex / se
    logp = sh - jnp.log(se)
    y = jnp.where(mask, jnp.log(probs + 1e-12) + gum_ref[...], -jnp.inf)
    ymax = jnp.max(y, axis=-1, keepdims=True)
    samp = jnp.min(jnp.where(y == ymax, iota_p, _LP), axis=-1, keepdims=True)
    onehot = iota_p == samp
    sp = jnp.sum(jnp.where(onehot, logp, 0.0), axis=-1)
    iota_r = lax.broadcasted_iota(jnp.int32, (_UB, _L), 1)
    sr = jnp.sum(jnp.where(iota_r == samp, rew_ref[...], 0.0), axis=-1)
    ue = ue_ref[...]
    gan_part = jnp.sum(sp * sr).reshape(1, 1)
    reg_part = (jnp.sum(jnp.where(mask, n_ref[...], 0.0))
                + jnp.sum(ue * ue)).reshape(1, 1)

    @pl.when(step == 0)
    def _():
        gan_ref[...] = jnp.zeros((1, 1), jnp.float32)
        reg_ref[...] = jnp.zeros((1, 1), jnp.float32)

    gan_ref[...] += gan_part
    reg_ref[...] += reg_part

    @pl.when(step == _GSTEPS - 1)
    def _():
        gan_ref[...] = -gan_ref[...] / _B
        reg_ref[...] = _REGS * 0.5 * reg_ref[...]


def _losses(d2, n, reward, gum, ue):
    return pl.pallas_call(
        _loss_body,
        grid=(_GSTEPS,),
        in_specs=[
            pl.BlockSpec((_UB, _LP), lambda i: (i, 0)),
            pl.BlockSpec((_UB, _LP), lambda i: (i, 0)),
            pl.BlockSpec((_UB, _L), lambda i: (i, 0)),
            pl.BlockSpec((_UB, _LP), lambda i: (i, 0)),
            pl.BlockSpec((_UB, _D), lambda i: (i, 0)),
        ],
        out_specs=[pl.BlockSpec((1, 1), lambda i: (0, 0)),
                   pl.BlockSpec((1, 1), lambda i: (0, 0))],
        out_shape=[jax.ShapeDtypeStruct((1, 1), jnp.float32),
                   jax.ShapeDtypeStruct((1, 1), jnp.float32)],
    )(d2, n, reward, gum, ue)


def kernel(user, items, reward, user_embedding, item_embedding,
           umlp_w, umlp_b, imlp_w, imlp_b):
    user = user.astype(jnp.int32)
    items_flat = items.astype(jnp.int32).reshape(_B * _L)
    t_item = _transform_table(item_embedding, imlp_w, imlp_b.reshape(1, _D))
    t_user = _transform_table(user_embedding, umlp_w, umlp_b.reshape(1, _D))
    d2, n, ue = _sc_dist(t_item, t_user, items_flat, user)
    d2 = d2.reshape(_B, _LP)
    n = n.reshape(_B, _LP)
    gum = jnp.concatenate(
        [jax.random.gumbel(jax.random.key(123), (_B, _L), jnp.float32),
         jnp.zeros((_B, _LP - _L), jnp.float32)], axis=1)
    gan, reg = _losses(d2, n, reward, gum, ue)
    return (gan.reshape(()), reg.reshape(()))
